# baseline (speedup 1.0000x reference)
import functools

import jax
import jax.numpy as jnp
from jax import lax
from jax.experimental import pallas as pl
from jax.experimental.pallas import tpu as pltpu
from jax.experimental.pallas import tpu_sc as plsc

N_GRAPHS = 16
N_SS = 512
SEQ = 1000
VOCAB = 26
D = 128

_N = 10000
_NP = 10016
_E = 160000
_EP = 163840
_PT = 5120
_P = 896
_CH = 64
_NCH = _PT // _CH


def _k2_body(xl_hbm, xr_hbm, src2_hbm, dst2_hbm, g1_hbm, g2_hbm,
             srcv, dstv, buf1, buf2, sem1, sem2):
    c = lax.axis_index("c")
    s = lax.axis_index("s")
    w = s * 2 + c
    nrow = _PT // 128
    pltpu.sync_copy(src2_hbm.at[pl.ds(w * nrow, nrow)], srcv)
    pltpu.sync_copy(dst2_hbm.at[pl.ds(w * nrow, nrow)], dstv)

    def body(k, carry):
        r = k // 2
        off = (k % 2) * _CH
        d1 = pltpu.async_copy(xl_hbm.at[srcv.at[r, pl.ds(off, _CH)]], buf1, sem1)
        d2 = pltpu.async_copy(xr_hbm.at[dstv.at[r, pl.ds(off, _CH)]], buf2, sem2)
        d1.wait()
        d2.wait()
        base = w * _PT + k * _CH
        pltpu.sync_copy(buf1, g1_hbm.at[pl.ds(base, _CH)])
        pltpu.sync_copy(buf2, g2_hbm.at[pl.ds(base, _CH)])
        return carry

    lax.fori_loop(0, _NCH, body, 0)


def _sc_gather_pair(xl_t, xr_t, src2, dst2):
    mesh = plsc.VectorSubcoreMesh(core_axis_name="c", subcore_axis_name="s")
    f = pl.kernel(
        _k2_body,
        out_type=(jax.ShapeDtypeStruct((_EP, _P), jnp.float32),
                  jax.ShapeDtypeStruct((_EP, _P), jnp.float32)),
        mesh=mesh,
        scratch_types=[
            pltpu.VMEM((_PT // 128, 128), jnp.int32),
            pltpu.VMEM((_PT // 128, 128), jnp.int32),
            pltpu.VMEM((_CH, _P), jnp.float32),
            pltpu.VMEM((_CH, _P), jnp.float32),
            pltpu.SemaphoreType.DMA,
            pltpu.SemaphoreType.DMA,
        ],
    )
    return f(xl_t, xr_t, src2, dst2)


def _ln(v, g, b):
    mu = jnp.mean(v, -1, keepdims=True)
    var = jnp.var(v, -1, keepdims=True)
    return (v - mu) / jnp.sqrt(var + 1e-5) * g + b


def _mha_self(t, L):
    B, Lq, Dm = t.shape
    H, dh = 2, Dm // 2
    Q = (t @ L['sa_wq'] + L['sa_bq']).reshape(B, Lq, H, dh).transpose(0, 2, 1, 3)
    K = (t @ L['sa_wk'] + L['sa_bk']).reshape(B, Lq, H, dh).transpose(0, 2, 1, 3)
    V = (t @ L['sa_wv'] + L['sa_bv']).reshape(B, Lq, H, dh).transpose(0, 2, 1, 3)
    s = jnp.einsum('bhqd,bhkd->bhqk', Q, K) / jnp.sqrt(float(dh))
    a = jax.nn.softmax(s, axis=-1)
    o = jnp.einsum('bhqk,bhkd->bhqd', a, V).transpose(0, 2, 1, 3).reshape(B, Lq, Dm)
    return o @ L['sa_wo'] + L['sa_bo']


def _final_mlp_body(xc_ref, w1_ref, b1_ref, w2_ref, b2_ref, w3_ref, b3_ref, out_ref):
    h1 = jnp.maximum(jnp.dot(xc_ref[...], w1_ref[...],
                             preferred_element_type=jnp.float32) + b1_ref[...], 0.0)
    h2 = jnp.maximum(jnp.dot(h1, w2_ref[...],
                             preferred_element_type=jnp.float32) + b2_ref[...], 0.0)
    out_ref[...] = jnp.dot(h2, w3_ref[...],
                           preferred_element_type=jnp.float32) + b3_ref[...]


def _final_mlp(xc, p):
    return pl.pallas_call(
        _final_mlp_body,
        out_shape=jax.ShapeDtypeStruct((xc.shape[0], 1), jnp.float32),
    )(xc, p['fc1_w'], p['fc1_b'][None, :], p['fc2_w'], p['fc2_b'][None, :],
      p['out_w'], p['out_b'][None, :])


def kernel(x, edge_index, batch, target, ss_feat, sas_feat, eds_contact, params):
    p = params
    N = x.shape[0]
    H, C = 10, 78
    src, dst = edge_index[0], edge_index[1]
    srcp = jnp.concatenate([src, jnp.zeros((_EP - _E,), src.dtype)])
    dstp = jnp.concatenate([dst, jnp.full((_EP - _E,), _N, dst.dtype)])
    src2 = srcp.reshape(_EP // 128, 128)
    dst2 = dstp.reshape(_EP // 128, 128)

    xl2 = x @ p['gat_wl']
    xr2 = x @ p['gat_wr']
    xl = xl2.reshape(N, H, C)
    xr = xr2.reshape(N, H, C)
    xl_t = jnp.pad(xl2, ((0, _NP - _N), (0, _P - H * C)))
    xr_t = jnp.pad(xr2, ((0, _NP - _N), (0, _P - H * C)))

    g1, g2 = _sc_gather_pair(xl_t, xr_t, src2, dst2)
    g1e = g1[:_E, :H * C].reshape(_E, H, C)
    g2e = g2[:_E, :H * C].reshape(_E, H, C)

    loops = jnp.arange(N, dtype=src.dtype)
    srcf = jnp.concatenate([src, loops])
    dstf = jnp.concatenate([dst, loops])
    xls = jnp.concatenate([g1e, xl], axis=0)
    xrd = jnp.concatenate([g2e, xr], axis=0)

    m = jax.nn.leaky_relu(xls + xrd, 0.2)
    e = jnp.einsum('ehc,hc->eh', m, p['gat_att'])
    emax = jax.ops.segment_max(e, dstf, num_segments=N)
    ex = jnp.exp(e - emax[dstf])
    den = jax.ops.segment_sum(ex, dstf, num_segments=N)
    alpha = ex / (den[dstf] + 1e-16)
    h = jax.ops.segment_sum(alpha[:, :, None] * xls, dstf, num_segments=N).reshape(N, H * C) + p['gat_b']
    h = jax.nn.relu(h)
    deg = jax.ops.segment_sum(jnp.ones((srcf.shape[0],), jnp.float32), dstf, num_segments=N)
    dinv = 1.0 / jnp.sqrt(jnp.clip(deg, 1.0))
    norm = dinv[srcf] * dinv[dstf]
    h2 = h @ p['gcn_w']
    h = jax.ops.segment_sum(norm[:, None] * h2[srcf], dstf, num_segments=N) + p['gcn_b']
    h = jax.nn.relu(h)

    gmax = jax.ops.segment_max(h, batch, num_segments=N_GRAPHS)
    gmax = jnp.where(jnp.isfinite(gmax), gmax, 0.0)
    gsum = jax.ops.segment_sum(h, batch, num_segments=N_GRAPHS)
    gcnt = jax.ops.segment_sum(jnp.ones((N,), jnp.float32), batch, num_segments=N_GRAPHS)
    gap = gsum / jnp.clip(gcnt, 1.0)[:, None]
    xg = jnp.concatenate([gmax, gap], axis=1)
    xg = jax.nn.relu(xg @ p['fc_g1_w'] + p['fc_g1_b'])
    xg = xg @ p['fc_g2_w'] + p['fc_g2_b']

    def pool512(v):
        mx = jnp.max(v, axis=0)
        mn = jnp.mean(v, axis=0)
        row = jnp.concatenate([mx, mn])[None, :]
        out = jnp.zeros((N_SS, 2 * v.shape[1]), jnp.float32)
        return out.at[N_SS - 1].set(row[0])

    ss = pool512(ss_feat) @ p['fc_g3_w'] + p['fc_g3_b']
    sas = pool512(sas_feat) @ p['fc_g4_w'] + p['fc_g4_b']
    eds = pool512(eds_contact) @ p['fc_g5_w'] + p['fc_g5_b']

    t = p['emb'][target]
    for L in p['dec']:
        a = _mha_self(t, L)
        t = _ln(t + a, L['ln1_g'], L['ln1_b'])
        ca = ((xg @ L['ca_wv'] + L['ca_bv']) @ L['ca_wo'] + L['ca_bo'])[:, None, :]
        t = _ln(t + ca, L['ln2_g'], L['ln2_b'])
        f = jax.nn.relu(t @ L['pf_w1'] + L['pf_b1']) @ L['pf_w2'] + L['pf_b2']
        t = _ln(t + f, L['ln3_g'], L['ln3_b'])
    conv = lax.conv_general_dilated(t, p['conv_w'], (1,), 'VALID',
                                    dimension_numbers=('NCH', 'OIH', 'NCH'))
    conv = conv + p['conv_b'][None, :, None]
    xt = conv.reshape(N_GRAPHS, 32 * 121) @ p['fc1_xt_w'] + p['fc1_xt_b']
    xg_p = jnp.zeros((N_SS, 128), jnp.float32).at[:N_GRAPHS].set(xg)
    xt_p = jnp.zeros((N_SS, 128), jnp.float32).at[:N_GRAPHS].set(xt)
    xc = jnp.concatenate([xg_p, xt_p, ss, sas, eds], axis=1)
    return _final_mlp(xc, p)


# speedup vs baseline: 3.1676x; 3.1676x over previous
import functools

import jax
import jax.numpy as jnp
from jax import lax
from jax.experimental import pallas as pl
from jax.experimental.pallas import tpu as pltpu
from jax.experimental.pallas import tpu_sc as plsc

N_GRAPHS = 16
N_SS = 512
SEQ = 1000
VOCAB = 26
D = 128

_N = 10000
_NP = 10016
_NS = 10240
_E = 160000
_EP = 163840
_PT = 5120
_P = 896
_CH = 64
_NCH = _PT // _CH


def _k2_body(xl_hbm, xr_hbm, src2_hbm, dst2_hbm, g1_hbm, g2_hbm,
             srcv, dstv, buf1, buf2, sem1, sem2):
    c = lax.axis_index("c")
    s = lax.axis_index("s")
    w = s * 2 + c
    nrow = _PT // 128
    pltpu.sync_copy(src2_hbm.at[pl.ds(w * nrow, nrow)], srcv)
    pltpu.sync_copy(dst2_hbm.at[pl.ds(w * nrow, nrow)], dstv)

    def body(k, carry):
        r = k // 2
        off = (k % 2) * _CH
        d1 = pltpu.async_copy(xl_hbm.at[srcv.at[r, pl.ds(off, _CH)]], buf1, sem1)
        d2 = pltpu.async_copy(xr_hbm.at[dstv.at[r, pl.ds(off, _CH)]], buf2, sem2)
        d1.wait()
        d2.wait()
        base = w * _PT + k * _CH
        pltpu.sync_copy(buf1, g1_hbm.at[pl.ds(base, _CH)])
        pltpu.sync_copy(buf2, g2_hbm.at[pl.ds(base, _CH)])
        return carry

    lax.fori_loop(0, _NCH, body, 0)


def _sc_gather_pair(xl_t, xr_t, src2, dst2):
    mesh = plsc.VectorSubcoreMesh(core_axis_name="c", subcore_axis_name="s")
    f = pl.kernel(
        _k2_body,
        out_type=(jax.ShapeDtypeStruct((_EP, _P), jnp.float32),
                  jax.ShapeDtypeStruct((_EP, _P), jnp.float32)),
        mesh=mesh,
        scratch_types=[
            pltpu.VMEM((_PT // 128, 128), jnp.int32),
            pltpu.VMEM((_PT // 128, 128), jnp.int32),
            pltpu.VMEM((_CH, _P), jnp.float32),
            pltpu.VMEM((_CH, _P), jnp.float32),
            pltpu.SemaphoreType.DMA,
            pltpu.SemaphoreType.DMA,
        ],
    )
    return f(xl_t, xr_t, src2, dst2)


_BE = 512


def _k3_body(g1_ref, g2_ref, a_ref, r_ref, z8_ref):
    sv = g1_ref[...] + g2_ref[...]
    m = jnp.where(sv >= 0, sv, 0.2 * sv)
    e = jnp.dot(m, a_ref[...], preferred_element_type=jnp.float32)
    w = jnp.exp(e)
    wx = jnp.dot(w, r_ref[...], preferred_element_type=jnp.float32)
    z = wx * sv
    for q in range(7):
        z8_ref[q] = z[:, 128 * q:128 * (q + 1)]
    z8_ref[7] = jnp.concatenate(
        [w, jnp.zeros((w.shape[0], 112), jnp.float32)], axis=1)


def _tc_edge_math(g1, g2, a_m, r_m):
    return pl.pallas_call(
        _k3_body,
        grid=(_EP // _BE,),
        in_specs=[
            pl.BlockSpec((_BE, _P), lambda i: (i, 0)),
            pl.BlockSpec((_BE, _P), lambda i: (i, 0)),
            pl.BlockSpec((_P, 16), lambda i: (0, 0)),
            pl.BlockSpec((16, _P), lambda i: (0, 0)),
        ],
        out_specs=pl.BlockSpec((8, _BE, 128), lambda i: (0, i, 0)),
        out_shape=jax.ShapeDtypeStruct((8, _EP, 128), jnp.float32),
    )(g1, g2, a_m, r_m)


def _k4_body(z8_hbm, dst2_hbm, part_hbm, dstv, buf, acc, sem):
    c = lax.axis_index("c")
    s = lax.axis_index("s")
    w = s * 2 + c

    pltpu.sync_copy(dst2_hbm.at[pl.ds(w * 40, 40)], dstv)

    def zero_row(i, carry):
        for j in range(8):
            buf[i, pl.ds(j * 16, 16)] = jnp.zeros((16,), jnp.float32)
        return carry

    for qp in range(8):
        lax.fori_loop(0, 128, zero_row, 0)
        for j in range(5):
            pltpu.sync_copy(buf, acc.at[pl.ds(s * 640 + j * 128, 128)])
        plsc.subcore_barrier()

        def chunk(k, carry):
            base = w * _PT + k * 128
            pltpu.async_copy(z8_hbm.at[qp, pl.ds(base, 128)], buf, sem).wait()
            pltpu.sync_copy(buf, acc.at[dstv.at[k]], add=True)
            return carry

        lax.fori_loop(0, 40, chunk, 0)
        plsc.subcore_barrier()
        pltpu.sync_copy(acc.at[pl.ds(s * 640, 640)],
                        part_hbm.at[c, qp, pl.ds(s * 640, 640)])
        plsc.subcore_barrier()


def _sc_scatter_z(z8, dst2):
    mesh = plsc.VectorSubcoreMesh(core_axis_name="c", subcore_axis_name="s")
    f = pl.kernel(
        _k4_body,
        out_type=jax.ShapeDtypeStruct((2, 8, _NS, 128), jnp.float32),
        mesh=mesh,
        scratch_types=[
            pltpu.VMEM((40, 128), jnp.int32),
            pltpu.VMEM((128, 128), jnp.float32),
            pltpu.VMEM_SHARED((_NS, 128), jnp.float32),
            pltpu.SemaphoreType.DMA,
        ],
    )
    return f(z8, dst2)


def _ln(v, g, b):
    mu = jnp.mean(v, -1, keepdims=True)
    var = jnp.var(v, -1, keepdims=True)
    return (v - mu) / jnp.sqrt(var + 1e-5) * g + b


def _mha_self(t, L):
    B, Lq, Dm = t.shape
    H, dh = 2, Dm // 2
    Q = (t @ L['sa_wq'] + L['sa_bq']).reshape(B, Lq, H, dh).transpose(0, 2, 1, 3)
    K = (t @ L['sa_wk'] + L['sa_bk']).reshape(B, Lq, H, dh).transpose(0, 2, 1, 3)
    V = (t @ L['sa_wv'] + L['sa_bv']).reshape(B, Lq, H, dh).transpose(0, 2, 1, 3)
    s = jnp.einsum('bhqd,bhkd->bhqk', Q, K) / jnp.sqrt(float(dh))
    a = jax.nn.softmax(s, axis=-1)
    o = jnp.einsum('bhqk,bhkd->bhqd', a, V).transpose(0, 2, 1, 3).reshape(B, Lq, Dm)
    return o @ L['sa_wo'] + L['sa_bo']


def _final_mlp_body(xc_ref, w1_ref, b1_ref, w2_ref, b2_ref, w3_ref, b3_ref, out_ref):
    h1 = jnp.maximum(jnp.dot(xc_ref[...], w1_ref[...],
                             preferred_element_type=jnp.float32) + b1_ref[...], 0.0)
    h2 = jnp.maximum(jnp.dot(h1, w2_ref[...],
                             preferred_element_type=jnp.float32) + b2_ref[...], 0.0)
    out_ref[...] = jnp.dot(h2, w3_ref[...],
                           preferred_element_type=jnp.float32) + b3_ref[...]


def _final_mlp(xc, p):
    return pl.pallas_call(
        _final_mlp_body,
        out_shape=jax.ShapeDtypeStruct((xc.shape[0], 1), jnp.float32),
    )(xc, p['fc1_w'], p['fc1_b'][None, :], p['fc2_w'], p['fc2_b'][None, :],
      p['out_w'], p['out_b'][None, :])


def kernel(x, edge_index, batch, target, ss_feat, sas_feat, eds_contact, params):
    p = params
    N = x.shape[0]
    H, C = 10, 78
    src, dst = edge_index[0], edge_index[1]
    srcp = jnp.concatenate([src, jnp.zeros((_EP - _E,), src.dtype)])
    dstp = jnp.concatenate([dst, jnp.full((_EP - _E,), _N, dst.dtype)])
    src2 = srcp.reshape(_EP // 128, 128)
    dst2 = dstp.reshape(_EP // 128, 128)

    xl2 = x @ p['gat_wl']
    xr2 = x @ p['gat_wr']
    xl = xl2.reshape(N, H, C)
    xr = xr2.reshape(N, H, C)
    xl_t = jnp.pad(xl2, ((0, _NP - _N), (0, _P - H * C)))
    xr_t = jnp.pad(xr2, ((0, _NP - _N), (0, _P - H * C)))

    g1, g2 = _sc_gather_pair(xl_t, xr_t, src2, dst2)

    att = p['gat_att']
    blk = jnp.zeros((H, C, H), jnp.float32).at[
        jnp.arange(H), :, jnp.arange(H)].set(att)
    a_m = jnp.pad(blk.reshape(H * C, H), ((0, _P - H * C), (0, 6)))
    rblk = jnp.zeros((H, H, C), jnp.float32).at[
        jnp.arange(H), jnp.arange(H), :].set(1.0)
    r_m = jnp.pad(rblk.reshape(H, H * C), ((0, 6), (0, _P - H * C)))

    z8 = _tc_edge_math(g1, g2, a_m, r_m)
    part = _sc_scatter_z(z8, dst2)
    acc = part[0][:, :_N] + part[1][:, :_N]
    acc896 = jnp.moveaxis(acc[:7], 0, 1).reshape(_N, _P)
    den_e = acc[7][:, :16]

    ws = jnp.exp(jnp.einsum('nhc,hc->nh',
                            jax.nn.leaky_relu(xl + xr, 0.2), att))
    den_tot10 = den_e[:, :H] + ws
    deg = den_e[:, 15] + 1.0
    out3 = (acc896[:, :H * C].reshape(N, H, C)
            - den_e[:, :H, None] * xr + ws[:, :, None] * xl)
    h = (out3 / (den_tot10[:, :, None] + 1e-16)).reshape(N, H * C) + p['gat_b']
    h = jax.nn.relu(h)

    loops = jnp.arange(N, dtype=src.dtype)
    srcf = jnp.concatenate([src, loops])
    dstf = jnp.concatenate([dst, loops])
    dinv = 1.0 / jnp.sqrt(jnp.clip(deg, 1.0))
    norm = dinv[srcf] * dinv[dstf]
    h2 = h @ p['gcn_w']
    h = jax.ops.segment_sum(norm[:, None] * h2[srcf], dstf, num_segments=N) + p['gcn_b']
    h = jax.nn.relu(h)

    gmax = jax.ops.segment_max(h, batch, num_segments=N_GRAPHS)
    gmax = jnp.where(jnp.isfinite(gmax), gmax, 0.0)
    gsum = jax.ops.segment_sum(h, batch, num_segments=N_GRAPHS)
    gcnt = jax.ops.segment_sum(jnp.ones((N,), jnp.float32), batch, num_segments=N_GRAPHS)
    gap = gsum / jnp.clip(gcnt, 1.0)[:, None]
    xg = jnp.concatenate([gmax, gap], axis=1)
    xg = jax.nn.relu(xg @ p['fc_g1_w'] + p['fc_g1_b'])
    xg = xg @ p['fc_g2_w'] + p['fc_g2_b']

    def pool512(v):
        mx = jnp.max(v, axis=0)
        mn = jnp.mean(v, axis=0)
        row = jnp.concatenate([mx, mn])[None, :]
        out = jnp.zeros((N_SS, 2 * v.shape[1]), jnp.float32)
        return out.at[N_SS - 1].set(row[0])

    ss = pool512(ss_feat) @ p['fc_g3_w'] + p['fc_g3_b']
    sas = pool512(sas_feat) @ p['fc_g4_w'] + p['fc_g4_b']
    eds = pool512(eds_contact) @ p['fc_g5_w'] + p['fc_g5_b']

    t = p['emb'][target]
    for L in p['dec']:
        a = _mha_self(t, L)
        t = _ln(t + a, L['ln1_g'], L['ln1_b'])
        ca = ((xg @ L['ca_wv'] + L['ca_bv']) @ L['ca_wo'] + L['ca_bo'])[:, None, :]
        t = _ln(t + ca, L['ln2_g'], L['ln2_b'])
        f = jax.nn.relu(t @ L['pf_w1'] + L['pf_b1']) @ L['pf_w2'] + L['pf_b2']
        t = _ln(t + f, L['ln3_g'], L['ln3_b'])
    conv = lax.conv_general_dilated(t, p['conv_w'], (1,), 'VALID',
                                    dimension_numbers=('NCH', 'OIH', 'NCH'))
    conv = conv + p['conv_b'][None, :, None]
    xt = conv.reshape(N_GRAPHS, 32 * 121) @ p['fc1_xt_w'] + p['fc1_xt_b']
    xg_p = jnp.zeros((N_SS, 128), jnp.float32).at[:N_GRAPHS].set(xg)
    xt_p = jnp.zeros((N_SS, 128), jnp.float32).at[:N_GRAPHS].set(xt)
    xc = jnp.concatenate([xg_p, xt_p, ss, sas, eds], axis=1)
    return _final_mlp(xc, p)


# speedup vs baseline: 5.3732x; 1.6963x over previous
import functools

import jax
import jax.numpy as jnp
from jax import lax
from jax.experimental import pallas as pl
from jax.experimental.pallas import tpu as pltpu
from jax.experimental.pallas import tpu_sc as plsc

N_GRAPHS = 16
N_SS = 512
SEQ = 1000
VOCAB = 26
D = 128

_N = 10000
_NP = 10016
_NS = 10240
_E = 160000
_EP = 163840
_PT = 5120
_P = 896
_CH = 64
_NCH = _PT // _CH


def _k2_body(xl_hbm, xr_hbm, src2_hbm, dst2_hbm, g1_hbm, g2_hbm,
             srcv, dstv, buf1, buf2, sem1, sem2):
    c = lax.axis_index("c")
    s = lax.axis_index("s")
    w = s * 2 + c
    nrow = _PT // 128
    pltpu.sync_copy(src2_hbm.at[pl.ds(w * nrow, nrow)], srcv)
    pltpu.sync_copy(dst2_hbm.at[pl.ds(w * nrow, nrow)], dstv)

    def body(k, carry):
        r = k // 2
        off = (k % 2) * _CH
        d1 = pltpu.async_copy(xl_hbm.at[srcv.at[r, pl.ds(off, _CH)]], buf1, sem1)
        d2 = pltpu.async_copy(xr_hbm.at[dstv.at[r, pl.ds(off, _CH)]], buf2, sem2)
        d1.wait()
        d2.wait()
        base = w * _PT + k * _CH
        pltpu.sync_copy(buf1, g1_hbm.at[pl.ds(base, _CH)])
        pltpu.sync_copy(buf2, g2_hbm.at[pl.ds(base, _CH)])
        return carry

    lax.fori_loop(0, _NCH, body, 0)


def _sc_gather_pair(xl_t, xr_t, src2, dst2):
    mesh = plsc.VectorSubcoreMesh(core_axis_name="c", subcore_axis_name="s")
    f = pl.kernel(
        _k2_body,
        out_type=(jax.ShapeDtypeStruct((_EP, _P), jnp.float32),
                  jax.ShapeDtypeStruct((_EP, _P), jnp.float32)),
        mesh=mesh,
        scratch_types=[
            pltpu.VMEM((_PT // 128, 128), jnp.int32),
            pltpu.VMEM((_PT // 128, 128), jnp.int32),
            pltpu.VMEM((_CH, _P), jnp.float32),
            pltpu.VMEM((_CH, _P), jnp.float32),
            pltpu.SemaphoreType.DMA,
            pltpu.SemaphoreType.DMA,
        ],
    )
    return f(xl_t, xr_t, src2, dst2)


_BE = 512


def _k3_body(g1_ref, g2_ref, a_ref, r_ref, z8_ref):
    sv = g1_ref[...] + g2_ref[...]
    m = jnp.where(sv >= 0, sv, 0.2 * sv)
    e = jnp.dot(m, a_ref[...], preferred_element_type=jnp.float32)
    w = jnp.exp(e)
    wx = jnp.dot(w, r_ref[...], preferred_element_type=jnp.float32)
    z = wx * sv
    for q in range(7):
        z8_ref[q] = z[:, 128 * q:128 * (q + 1)]
    z8_ref[7] = jnp.concatenate(
        [w, jnp.zeros((w.shape[0], 112), jnp.float32)], axis=1)


def _tc_edge_math(g1, g2, a_m, r_m):
    return pl.pallas_call(
        _k3_body,
        grid=(_EP // _BE,),
        in_specs=[
            pl.BlockSpec((_BE, _P), lambda i: (i, 0)),
            pl.BlockSpec((_BE, _P), lambda i: (i, 0)),
            pl.BlockSpec((_P, 16), lambda i: (0, 0)),
            pl.BlockSpec((16, _P), lambda i: (0, 0)),
        ],
        out_specs=pl.BlockSpec((8, _BE, 128), lambda i: (0, i, 0)),
        out_shape=jax.ShapeDtypeStruct((8, _EP, 128), jnp.float32),
    )(g1, g2, a_m, r_m)


def _k4_body(z8_hbm, dst2_hbm, part_hbm, dstv, buf, acc, sem):
    c = lax.axis_index("c")
    s = lax.axis_index("s")
    w = s * 2 + c

    pltpu.sync_copy(dst2_hbm.at[pl.ds(w * 40, 40)], dstv)

    def zero_row(i, carry):
        for j in range(8):
            buf[i, pl.ds(j * 16, 16)] = jnp.zeros((16,), jnp.float32)
        return carry

    for qp in range(8):
        lax.fori_loop(0, 128, zero_row, 0)
        for j in range(5):
            pltpu.sync_copy(buf, acc.at[pl.ds(s * 640 + j * 128, 128)])
        plsc.subcore_barrier()

        def chunk(k, carry):
            base = w * _PT + k * 128
            pltpu.async_copy(z8_hbm.at[qp, pl.ds(base, 128)], buf, sem).wait()
            pltpu.sync_copy(buf, acc.at[dstv.at[k]], add=True)
            return carry

        lax.fori_loop(0, 40, chunk, 0)
        plsc.subcore_barrier()
        pltpu.sync_copy(acc.at[pl.ds(s * 640, 640)],
                        part_hbm.at[c, qp, pl.ds(s * 640, 640)])
        plsc.subcore_barrier()


def _sc_scatter_z(z8, dst2):
    mesh = plsc.VectorSubcoreMesh(core_axis_name="c", subcore_axis_name="s")
    f = pl.kernel(
        _k4_body,
        out_type=jax.ShapeDtypeStruct((2, 8, _NS, 128), jnp.float32),
        mesh=mesh,
        scratch_types=[
            pltpu.VMEM((40, 128), jnp.int32),
            pltpu.VMEM((128, 128), jnp.float32),
            pltpu.VMEM_SHARED((_NS, 128), jnp.float32),
            pltpu.SemaphoreType.DMA,
        ],
    )
    return f(z8, dst2)


def _k6_body(qflat_hbm, src7_hbm, dst2_hbm, part_hbm, srcv, dstv, buf, acc, sem):
    c = lax.axis_index("c")
    s = lax.axis_index("s")
    w = s * 2 + c

    pltpu.sync_copy(dst2_hbm.at[pl.ds(w * 40, 40)], dstv)

    def zero_row(i, carry):
        for j in range(8):
            buf[i, pl.ds(j * 16, 16)] = jnp.zeros((16,), jnp.float32)
        return carry

    for qp in range(7):
        pltpu.sync_copy(src7_hbm.at[qp, pl.ds(w * 40, 40)], srcv)
        lax.fori_loop(0, 128, zero_row, 0)
        for j in range(5):
            pltpu.sync_copy(buf, acc.at[pl.ds(s * 640 + j * 128, 128)])
        plsc.subcore_barrier()

        def chunk(k, carry):
            pltpu.async_copy(qflat_hbm.at[srcv.at[k]], buf, sem).wait()
            pltpu.sync_copy(buf, acc.at[dstv.at[k]], add=True)
            return carry

        lax.fori_loop(0, 40, chunk, 0)
        plsc.subcore_barrier()
        pltpu.sync_copy(acc.at[pl.ds(s * 640, 640)],
                        part_hbm.at[c, qp, pl.ds(s * 640, 640)])
        plsc.subcore_barrier()


def _sc_gcn_agg(qflat, src7, dst2):
    mesh = plsc.VectorSubcoreMesh(core_axis_name="c", subcore_axis_name="s")
    f = pl.kernel(
        _k6_body,
        out_type=jax.ShapeDtypeStruct((2, 7, _NS, 128), jnp.float32),
        mesh=mesh,
        scratch_types=[
            pltpu.VMEM((40, 128), jnp.int32),
            pltpu.VMEM((40, 128), jnp.int32),
            pltpu.VMEM((128, 128), jnp.float32),
            pltpu.VMEM_SHARED((_NS, 128), jnp.float32),
            pltpu.SemaphoreType.DMA,
        ],
    )
    return f(qflat, src7, dst2)


def _ln(v, g, b):
    mu = jnp.mean(v, -1, keepdims=True)
    var = jnp.var(v, -1, keepdims=True)
    return (v - mu) / jnp.sqrt(var + 1e-5) * g + b


def _mha_self(t, L):
    B, Lq, Dm = t.shape
    H, dh = 2, Dm // 2
    Q = (t @ L['sa_wq'] + L['sa_bq']).reshape(B, Lq, H, dh).transpose(0, 2, 1, 3)
    K = (t @ L['sa_wk'] + L['sa_bk']).reshape(B, Lq, H, dh).transpose(0, 2, 1, 3)
    V = (t @ L['sa_wv'] + L['sa_bv']).reshape(B, Lq, H, dh).transpose(0, 2, 1, 3)
    s = jnp.einsum('bhqd,bhkd->bhqk', Q, K) / jnp.sqrt(float(dh))
    a = jax.nn.softmax(s, axis=-1)
    o = jnp.einsum('bhqk,bhkd->bhqd', a, V).transpose(0, 2, 1, 3).reshape(B, Lq, Dm)
    return o @ L['sa_wo'] + L['sa_bo']


def _final_mlp_body(xc_ref, w1_ref, b1_ref, w2_ref, b2_ref, w3_ref, b3_ref, out_ref):
    h1 = jnp.maximum(jnp.dot(xc_ref[...], w1_ref[...],
                             preferred_element_type=jnp.float32) + b1_ref[...], 0.0)
    h2 = jnp.maximum(jnp.dot(h1, w2_ref[...],
                             preferred_element_type=jnp.float32) + b2_ref[...], 0.0)
    out_ref[...] = jnp.dot(h2, w3_ref[...],
                           preferred_element_type=jnp.float32) + b3_ref[...]


def _final_mlp(xc, p):
    return pl.pallas_call(
        _final_mlp_body,
        out_shape=jax.ShapeDtypeStruct((xc.shape[0], 1), jnp.float32),
    )(xc, p['fc1_w'], p['fc1_b'][None, :], p['fc2_w'], p['fc2_b'][None, :],
      p['out_w'], p['out_b'][None, :])


def kernel(x, edge_index, batch, target, ss_feat, sas_feat, eds_contact, params):
    p = params
    N = x.shape[0]
    H, C = 10, 78
    src, dst = edge_index[0], edge_index[1]
    srcp = jnp.concatenate([src, jnp.zeros((_EP - _E,), src.dtype)])
    dstp = jnp.concatenate([dst, jnp.full((_EP - _E,), _N, dst.dtype)])
    src2 = srcp.reshape(_EP // 128, 128)
    dst2 = dstp.reshape(_EP // 128, 128)

    xl2 = x @ p['gat_wl']
    xr2 = x @ p['gat_wr']
    xl = xl2.reshape(N, H, C)
    xr = xr2.reshape(N, H, C)
    xl_t = jnp.pad(xl2, ((0, _NP - _N), (0, _P - H * C)))
    xr_t = jnp.pad(xr2, ((0, _NP - _N), (0, _P - H * C)))

    g1, g2 = _sc_gather_pair(xl_t, xr_t, src2, dst2)

    att = p['gat_att']
    blk = jnp.zeros((H, C, H), jnp.float32).at[
        jnp.arange(H), :, jnp.arange(H)].set(att)
    a_m = jnp.pad(blk.reshape(H * C, H), ((0, _P - H * C), (0, 6)))
    rblk = jnp.zeros((H, H, C), jnp.float32).at[
        jnp.arange(H), jnp.arange(H), :].set(1.0)
    r_m = jnp.pad(rblk.reshape(H, H * C), ((0, 6), (0, _P - H * C)))

    z8 = _tc_edge_math(g1, g2, a_m, r_m)
    part = _sc_scatter_z(z8, dst2)
    acc = part[0][:, :_N] + part[1][:, :_N]
    acc896 = jnp.moveaxis(acc[:7], 0, 1).reshape(_N, _P)
    den_e = acc[7][:, :16]

    ws = jnp.exp(jnp.einsum('nhc,hc->nh',
                            jax.nn.leaky_relu(xl + xr, 0.2), att))
    den_tot10 = den_e[:, :H] + ws
    deg = den_e[:, 15] + 1.0
    out3 = (acc896[:, :H * C].reshape(N, H, C)
            - den_e[:, :H, None] * xr + ws[:, :, None] * xl)
    h = (out3 / (den_tot10[:, :, None] + 1e-16)).reshape(N, H * C) + p['gat_b']
    h = jax.nn.relu(h)

    dinv = 1.0 / jnp.sqrt(jnp.clip(deg, 1.0))
    h2 = h @ p['gcn_w']
    q896 = jnp.pad(dinv[:, None] * h2, ((0, _NP - _N), (0, _P - H * C)))
    qflat = jnp.moveaxis(q896.reshape(_NP, 7, 128), 1, 0).reshape(7 * _NP, 128)
    src7 = (srcp[None, :] + (_NP * jnp.arange(7, dtype=srcp.dtype))[:, None]
            ).reshape(7, _EP // 128, 128)
    part2 = _sc_gcn_agg(qflat, src7, dst2)
    agg = part2[0][:, :_N] + part2[1][:, :_N]
    agg896 = jnp.moveaxis(agg, 0, 1).reshape(_N, _P)
    h = dinv[:, None] * agg896[:, :H * C] + dinv[:, None] ** 2 * h2 + p['gcn_b']
    h = jax.nn.relu(h)

    gmax = jax.ops.segment_max(h, batch, num_segments=N_GRAPHS)
    gmax = jnp.where(jnp.isfinite(gmax), gmax, 0.0)
    gsum = jax.ops.segment_sum(h, batch, num_segments=N_GRAPHS)
    gcnt = jax.ops.segment_sum(jnp.ones((N,), jnp.float32), batch, num_segments=N_GRAPHS)
    gap = gsum / jnp.clip(gcnt, 1.0)[:, None]
    xg = jnp.concatenate([gmax, gap], axis=1)
    xg = jax.nn.relu(xg @ p['fc_g1_w'] + p['fc_g1_b'])
    xg = xg @ p['fc_g2_w'] + p['fc_g2_b']

    def pool512(v):
        mx = jnp.max(v, axis=0)
        mn = jnp.mean(v, axis=0)
        row = jnp.concatenate([mx, mn])[None, :]
        out = jnp.zeros((N_SS, 2 * v.shape[1]), jnp.float32)
        return out.at[N_SS - 1].set(row[0])

    ss = pool512(ss_feat) @ p['fc_g3_w'] + p['fc_g3_b']
    sas = pool512(sas_feat) @ p['fc_g4_w'] + p['fc_g4_b']
    eds = pool512(eds_contact) @ p['fc_g5_w'] + p['fc_g5_b']

    t = p['emb'][target]
    for L in p['dec']:
        a = _mha_self(t, L)
        t = _ln(t + a, L['ln1_g'], L['ln1_b'])
        ca = ((xg @ L['ca_wv'] + L['ca_bv']) @ L['ca_wo'] + L['ca_bo'])[:, None, :]
        t = _ln(t + ca, L['ln2_g'], L['ln2_b'])
        f = jax.nn.relu(t @ L['pf_w1'] + L['pf_b1']) @ L['pf_w2'] + L['pf_b2']
        t = _ln(t + f, L['ln3_g'], L['ln3_b'])
    conv = lax.conv_general_dilated(t, p['conv_w'], (1,), 'VALID',
                                    dimension_numbers=('NCH', 'OIH', 'NCH'))
    conv = conv + p['conv_b'][None, :, None]
    xt = conv.reshape(N_GRAPHS, 32 * 121) @ p['fc1_xt_w'] + p['fc1_xt_b']
    xg_p = jnp.zeros((N_SS, 128), jnp.float32).at[:N_GRAPHS].set(xg)
    xt_p = jnp.zeros((N_SS, 128), jnp.float32).at[:N_GRAPHS].set(xt)
    xc = jnp.concatenate([xg_p, xt_p, ss, sas, eds], axis=1)
    return _final_mlp(xc, p)


# speedup vs baseline: 7.0018x; 1.3031x over previous
import jax
import jax.numpy as jnp
from jax import lax
from jax.experimental import pallas as pl
from jax.experimental.pallas import tpu as pltpu
from jax.experimental.pallas import tpu_sc as plsc

N_GRAPHS = 16
N_SS = 512
SEQ = 1000
VOCAB = 26

_N = 10000
_NP = 10240
_E = 160000
_EP = 163840
_PT = 5120
_P = 896
_CH = 64
_NCH = _PT // _CH
_BN = 640
_BE = 512


def _k2_body(xl_hbm, xr_hbm, src2_hbm, dst2_hbm, g1_hbm, g2_hbm,
             srcv, dstv, buf1, buf2, sem1, sem2):
    c = lax.axis_index("c")
    s = lax.axis_index("s")
    w = s * 2 + c
    nrow = _PT // 128
    pltpu.sync_copy(src2_hbm.at[pl.ds(w * nrow, nrow)], srcv)
    pltpu.sync_copy(dst2_hbm.at[pl.ds(w * nrow, nrow)], dstv)

    def body(k, carry):
        r = k // 2
        off = (k % 2) * _CH
        d1 = pltpu.async_copy(xl_hbm.at[srcv.at[r, pl.ds(off, _CH)]], buf1, sem1)
        d2 = pltpu.async_copy(xr_hbm.at[dstv.at[r, pl.ds(off, _CH)]], buf2, sem2)
        d1.wait()
        d2.wait()
        base = w * _PT + k * _CH
        pltpu.sync_copy(buf1, g1_hbm.at[pl.ds(base, _CH)])
        pltpu.sync_copy(buf2, g2_hbm.at[pl.ds(base, _CH)])
        return carry

    lax.fori_loop(0, _NCH, body, 0)


def _sc_gather_pair(xl_t, xr_t, src2, dst2):
    mesh = plsc.VectorSubcoreMesh(core_axis_name="c", subcore_axis_name="s")
    f = pl.kernel(
        _k2_body,
        out_type=(jax.ShapeDtypeStruct((_EP, _P), jnp.float32),
                  jax.ShapeDtypeStruct((_EP, _P), jnp.float32)),
        mesh=mesh,
        scratch_types=[
            pltpu.VMEM((_PT // 128, 128), jnp.int32),
            pltpu.VMEM((_PT // 128, 128), jnp.int32),
            pltpu.VMEM((_CH, _P), jnp.float32),
            pltpu.VMEM((_CH, _P), jnp.float32),
            pltpu.SemaphoreType.DMA,
            pltpu.SemaphoreType.DMA,
        ],
    )
    return f(xl_t, xr_t, src2, dst2)


def _k3_body(g1_ref, g2_ref, a_ref, r_ref, z8_ref):
    sv = g1_ref[...] + g2_ref[...]
    m = jnp.where(sv >= 0, sv, 0.2 * sv)
    e = jnp.dot(m, a_ref[...], preferred_element_type=jnp.float32)
    w = jnp.exp(e)
    wx = jnp.dot(w, r_ref[...], preferred_element_type=jnp.float32)
    z = wx * sv
    for q in range(7):
        z8_ref[q] = z[:, 128 * q:128 * (q + 1)]
    z8_ref[7] = jnp.concatenate(
        [w, jnp.zeros((w.shape[0], 112), jnp.float32)], axis=1)


def _tc_edge_math(g1, g2, a_m, r_m):
    return pl.pallas_call(
        _k3_body,
        grid=(_EP // _BE,),
        in_specs=[
            pl.BlockSpec((_BE, _P), lambda i: (i, 0)),
            pl.BlockSpec((_BE, _P), lambda i: (i, 0)),
            pl.BlockSpec((_P, 16), lambda i: (0, 0)),
            pl.BlockSpec((16, _P), lambda i: (0, 0)),
        ],
        out_specs=pl.BlockSpec((8, _BE, 128), lambda i: (0, i, 0)),
        out_shape=jax.ShapeDtypeStruct((8, _EP, 128), jnp.float32),
    )(g1, g2, a_m, r_m)


def _k4_body(z8_hbm, dst2_hbm, part_hbm, dstv, buf, acc, sem):
    c = lax.axis_index("c")
    s = lax.axis_index("s")
    w = s * 2 + c

    pltpu.sync_copy(dst2_hbm.at[pl.ds(w * 40, 40)], dstv)

    def zero_row(i, carry):
        for j in range(8):
            buf[i, pl.ds(j * 16, 16)] = jnp.zeros((16,), jnp.float32)
        return carry

    for qp in range(8):
        lax.fori_loop(0, 128, zero_row, 0)
        for j in range(5):
            pltpu.sync_copy(buf, acc.at[pl.ds(s * 640 + j * 128, 128)])
        plsc.subcore_barrier()

        def chunk(k, carry):
            base = w * _PT + k * 128
            pltpu.async_copy(z8_hbm.at[qp, pl.ds(base, 128)], buf, sem).wait()
            pltpu.sync_copy(buf, acc.at[dstv.at[k]], add=True)
            return carry

        lax.fori_loop(0, 40, chunk, 0)
        plsc.subcore_barrier()
        pltpu.sync_copy(acc.at[pl.ds(s * 640, 640)],
                        part_hbm.at[c, qp, pl.ds(s * 640, 640)])
        plsc.subcore_barrier()


def _sc_scatter_z(z8, dst2):
    mesh = plsc.VectorSubcoreMesh(core_axis_name="c", subcore_axis_name="s")
    f = pl.kernel(
        _k4_body,
        out_type=jax.ShapeDtypeStruct((2, 8, _NP, 128), jnp.float32),
        mesh=mesh,
        scratch_types=[
            pltpu.VMEM((40, 128), jnp.int32),
            pltpu.VMEM((128, 128), jnp.float32),
            pltpu.VMEM_SHARED((_NP, 128), jnp.float32),
            pltpu.SemaphoreType.DMA,
        ],
    )
    return f(z8, dst2)


def _k6_body(qflat_hbm, src7_hbm, dst2_hbm, part_hbm, srcv, dstv, buf, acc, sem):
    c = lax.axis_index("c")
    s = lax.axis_index("s")
    w = s * 2 + c

    pltpu.sync_copy(dst2_hbm.at[pl.ds(w * 40, 40)], dstv)

    def zero_row(i, carry):
        for j in range(8):
            buf[i, pl.ds(j * 16, 16)] = jnp.zeros((16,), jnp.float32)
        return carry

    for qp in range(7):
        pltpu.sync_copy(src7_hbm.at[qp, pl.ds(w * 40, 40)], srcv)
        lax.fori_loop(0, 128, zero_row, 0)
        for j in range(5):
            pltpu.sync_copy(buf, acc.at[pl.ds(s * 640 + j * 128, 128)])
        plsc.subcore_barrier()

        def chunk(k, carry):
            pltpu.async_copy(qflat_hbm.at[srcv.at[k]], buf, sem).wait()
            pltpu.sync_copy(buf, acc.at[dstv.at[k]], add=True)
            return carry

        lax.fori_loop(0, 40, chunk, 0)
        plsc.subcore_barrier()
        pltpu.sync_copy(acc.at[pl.ds(s * 640, 640)],
                        part_hbm.at[c, qp, pl.ds(s * 640, 640)])
        plsc.subcore_barrier()


def _sc_gcn_agg(qflat, src7, dst2):
    mesh = plsc.VectorSubcoreMesh(core_axis_name="c", subcore_axis_name="s")
    f = pl.kernel(
        _k6_body,
        out_type=jax.ShapeDtypeStruct((2, 7, _NP, 128), jnp.float32),
        mesh=mesh,
        scratch_types=[
            pltpu.VMEM((40, 128), jnp.int32),
            pltpu.VMEM((40, 128), jnp.int32),
            pltpu.VMEM((128, 128), jnp.float32),
            pltpu.VMEM_SHARED((_NP, 128), jnp.float32),
            pltpu.SemaphoreType.DMA,
        ],
    )
    return f(qflat, src7, dst2)


def _k1_body(x_ref, wl_ref, wr_ref, a_ref, xl_ref, xr_ref, ws_ref):
    xb = x_ref[...]
    xl = jnp.dot(xb, wl_ref[...], preferred_element_type=jnp.float32)
    xr = jnp.dot(xb, wr_ref[...], preferred_element_type=jnp.float32)
    sv = xl + xr
    m = jnp.where(sv >= 0, sv, 0.2 * sv)
    ws_ref[...] = jnp.exp(jnp.dot(m, a_ref[...],
                                  preferred_element_type=jnp.float32))
    xl_ref[...] = xl
    xr_ref[...] = xr


def _tc_project(x_pad, wl_pad, wr_pad, a_m):
    return pl.pallas_call(
        _k1_body,
        grid=(_NP // _BN,),
        in_specs=[
            pl.BlockSpec((_BN, 78), lambda i: (i, 0)),
            pl.BlockSpec((78, _P), lambda i: (0, 0)),
            pl.BlockSpec((78, _P), lambda i: (0, 0)),
            pl.BlockSpec((_P, 16), lambda i: (0, 0)),
        ],
        out_specs=[
            pl.BlockSpec((_BN, _P), lambda i: (i, 0)),
            pl.BlockSpec((_BN, _P), lambda i: (i, 0)),
            pl.BlockSpec((_BN, 16), lambda i: (i, 0)),
        ],
        out_shape=[
            jax.ShapeDtypeStruct((_NP, _P), jnp.float32),
            jax.ShapeDtypeStruct((_NP, _P), jnp.float32),
            jax.ShapeDtypeStruct((_NP, 16), jnp.float32),
        ],
    )(x_pad, wl_pad, wr_pad, a_m)


def _k5_body(part_ref, xl_ref, xr_ref, ws_ref, r_ref, gatb_ref, pmask_ref,
             gcnw_ref, gcnb_ref, q3_ref, st_ref, dv_ref):
    pb = part_ref[...]
    acc = jnp.concatenate([pb[0, q] + pb[1, q] for q in range(7)], axis=1)
    den_e = (pb[0, 7] + pb[1, 7])[:, :16]
    ws = ws_ref[...]
    den_tot = den_e + ws
    rm = r_ref[...]
    corr = (jnp.dot(ws, rm, preferred_element_type=jnp.float32) * xl_ref[...]
            - jnp.dot(den_e, rm, preferred_element_type=jnp.float32) * xr_ref[...])
    denx = (jnp.dot(den_tot, rm, preferred_element_type=jnp.float32)
            + pmask_ref[...] + 1e-16)
    h = jnp.maximum((acc + corr) / denx + gatb_ref[...], 0.0)
    h2 = jnp.dot(h, gcnw_ref[...], preferred_element_type=jnp.float32)
    deg = den_tot[:, 15:16]
    dinv = lax.rsqrt(jnp.maximum(deg, 1.0))
    q = dinv * h2
    for qp in range(7):
        q3_ref[qp] = q[:, 128 * qp:128 * (qp + 1)]
    st_ref[...] = dinv * dinv * h2 + gcnb_ref[...]
    dv_ref[...] = dinv * jnp.ones((1, 8), jnp.float32)


def _tc_gat_assemble(part, xl_t, xr_t, wself, r_m, gatb, pmask, gcnw, gcnb):
    return pl.pallas_call(
        _k5_body,
        grid=(_NP // _BN,),
        in_specs=[
            pl.BlockSpec((2, 8, _BN, 128), lambda i: (0, 0, i, 0)),
            pl.BlockSpec((_BN, _P), lambda i: (i, 0)),
            pl.BlockSpec((_BN, _P), lambda i: (i, 0)),
            pl.BlockSpec((_BN, 16), lambda i: (i, 0)),
            pl.BlockSpec((16, _P), lambda i: (0, 0)),
            pl.BlockSpec((1, _P), lambda i: (0, 0)),
            pl.BlockSpec((1, _P), lambda i: (0, 0)),
            pl.BlockSpec((_P, _P), lambda i: (0, 0)),
            pl.BlockSpec((1, _P), lambda i: (0, 0)),
        ],
        out_specs=[
            pl.BlockSpec((7, _BN, 128), lambda i: (0, i, 0)),
            pl.BlockSpec((_BN, _P), lambda i: (i, 0)),
            pl.BlockSpec((_BN, 8), lambda i: (i, 0)),
        ],
        out_shape=[
            jax.ShapeDtypeStruct((7, _NP, 128), jnp.float32),
            jax.ShapeDtypeStruct((_NP, _P), jnp.float32),
            jax.ShapeDtypeStruct((_NP, 8), jnp.float32),
        ],
    )(part, xl_t, xr_t, wself, r_m, gatb, pmask, gcnw, gcnb)


def _k7_body(part2_ref, st_ref, dv_ref, bf_ref, gmax_ref, gsum_ref, cnt_ref):
    pb = part2_ref[...]
    agg = jnp.concatenate([pb[0, q] + pb[1, q] for q in range(7)], axis=1)
    hg = jnp.maximum(dv_ref[...][:, :1] * agg + st_ref[...], 0.0)
    bcol = bf_ref[...]
    gm, gs, ct = [], [], []
    for g in range(N_GRAPHS):
        mk = (bcol == float(g)).astype(jnp.float32)
        sel = hg * mk
        gm.append(jnp.max(sel, axis=0, keepdims=True))
        gs.append(jnp.sum(sel, axis=0, keepdims=True))
        ct.append(jnp.sum(mk, axis=0, keepdims=True))
    gm = jnp.concatenate(gm, axis=0)
    gs = jnp.concatenate(gs, axis=0)
    ct = jnp.concatenate(ct, axis=0) * jnp.ones((1, 128), jnp.float32)
    first = pl.program_id(0) == 0
    gmax_ref[...] = jnp.where(first, gm, jnp.maximum(gmax_ref[...], gm))
    gsum_ref[...] = jnp.where(first, gs, gsum_ref[...] + gs)
    cnt_ref[...] = jnp.where(first, ct, cnt_ref[...] + ct)


def _tc_pool(part2, st, dv, batchf):
    return pl.pallas_call(
        _k7_body,
        grid=(_NP // _BN,),
        in_specs=[
            pl.BlockSpec((2, 7, _BN, 128), lambda i: (0, 0, i, 0)),
            pl.BlockSpec((_BN, _P), lambda i: (i, 0)),
            pl.BlockSpec((_BN, 8), lambda i: (i, 0)),
            pl.BlockSpec((_BN, 1), lambda i: (i, 0)),
        ],
        out_specs=[
            pl.BlockSpec((N_GRAPHS, _P), lambda i: (0, 0)),
            pl.BlockSpec((N_GRAPHS, _P), lambda i: (0, 0)),
            pl.BlockSpec((N_GRAPHS, 128), lambda i: (0, 0)),
        ],
        out_shape=[
            jax.ShapeDtypeStruct((N_GRAPHS, _P), jnp.float32),
            jax.ShapeDtypeStruct((N_GRAPHS, _P), jnp.float32),
            jax.ShapeDtypeStruct((N_GRAPHS, 128), jnp.float32),
        ],
    )(part2, st, dv, batchf)


def _k7b_body(gmax_ref, gsum_ref, cnt_ref, w1_ref, b1_ref, w2_ref, b2_ref,
              xg_ref):
    gap = gsum_ref[...] / jnp.maximum(cnt_ref[...][:, :1], 1.0)
    xcat = jnp.concatenate([gmax_ref[...], gap], axis=1)
    h1 = jnp.maximum(jnp.dot(xcat, w1_ref[...],
                             preferred_element_type=jnp.float32) + b1_ref[...], 0.0)
    xg_ref[...] = jnp.dot(h1, w2_ref[...],
                          preferred_element_type=jnp.float32) + b2_ref[...]


def _tc_graph_head(gmax, gsum, cnt, w1p, b1, w2, b2):
    return pl.pallas_call(
        _k7b_body,
        out_shape=jax.ShapeDtypeStruct((N_GRAPHS, 128), jnp.float32),
    )(gmax, gsum, cnt, w1p, b1, w2, b2)


def _ln_p(v, g, b):
    mu = jnp.mean(v, axis=-1, keepdims=True)
    d = v - mu
    var = jnp.mean(d * d, axis=-1, keepdims=True)
    return d * lax.rsqrt(var + 1e-5) * g + b


def _k8_body(tgt_ref, xg_ref, emb_ref,
             saq_ref, sak_ref, sav_ref, sao_ref,
             sabq_ref, sabk_ref, sabv_ref, sabo_ref,
             ln1g_ref, ln1b_ref, ln2g_ref, ln2b_ref, ln3g_ref, ln3b_ref,
             cawv_ref, cabv_ref, cawo_ref, cabo_ref,
             pfw1_ref, pfb1_ref, pfw2_ref, pfb2_ref,
             cw_ref, cb_ref, conv_ref):
    tgt = tgt_ref[...].reshape(1, SEQ)
    ohT = (tgt == lax.broadcasted_iota(jnp.int32, (VOCAB, 1), 0)
           ).astype(jnp.float32)
    t = lax.dot_general(ohT, emb_ref[...], (((0,), (0,)), ((), ())),
                        preferred_element_type=jnp.float32)
    xgr = xg_ref[...].reshape(1, 128)
    for i in range(3):
        q = jnp.dot(t, saq_ref[i], preferred_element_type=jnp.float32) + sabq_ref[i]
        k = jnp.dot(t, sak_ref[i], preferred_element_type=jnp.float32) + sabk_ref[i]
        v = jnp.dot(t, sav_ref[i], preferred_element_type=jnp.float32) + sabv_ref[i]
        outs = []
        for hh in range(2):
            qh = q[:, 64 * hh:64 * hh + 64]
            kh = k[:, 64 * hh:64 * hh + 64]
            vh = v[:, 64 * hh:64 * hh + 64]
            sc = lax.dot_general(qh, kh, (((1,), (1,)), ((), ())),
                                 preferred_element_type=jnp.float32) * 0.125
            mx = jnp.max(sc, axis=-1, keepdims=True)
            pe = jnp.exp(sc - mx)
            aw = pe / jnp.sum(pe, axis=-1, keepdims=True)
            outs.append(jnp.dot(aw, vh, preferred_element_type=jnp.float32))
        o = jnp.dot(jnp.concatenate(outs, axis=1), sao_ref[i],
                    preferred_element_type=jnp.float32) + sabo_ref[i]
        t = _ln_p(t + o, ln1g_ref[i], ln1b_ref[i])
        ca = jnp.dot(jnp.dot(xgr, cawv_ref[i],
                             preferred_element_type=jnp.float32) + cabv_ref[i],
                     cawo_ref[i], preferred_element_type=jnp.float32) + cabo_ref[i]
        t = _ln_p(t + ca, ln2g_ref[i], ln2b_ref[i])
        f = jnp.dot(jnp.maximum(jnp.dot(t, pfw1_ref[i],
                                        preferred_element_type=jnp.float32)
                                + pfb1_ref[i], 0.0),
                    pfw2_ref[i], preferred_element_type=jnp.float32) + pfb2_ref[i]
        t = _ln_p(t + f, ln3g_ref[i], ln3b_ref[i])
    acc = cb_ref[...] * jnp.ones((1, 121), jnp.float32)
    for k8 in range(8):
        acc = acc + lax.dot_general(
            cw_ref[k8], t[:, k8:k8 + 121], (((0,), (0,)), ((), ())),
            preferred_element_type=jnp.float32)
    conv_ref[0] = acc


def _full_spec(shape):
    nd = len(shape)
    return pl.BlockSpec(shape, lambda g, _nd=nd: (0,) * _nd)


def _tc_decoder(tgt3, xg, emb, sd, cw8, cb):
    return pl.pallas_call(
        _k8_body,
        grid=(N_GRAPHS,),
        in_specs=[
            pl.BlockSpec((1, 1, SEQ), lambda g: (g, 0, 0)),
            pl.BlockSpec((1, 1, 128), lambda g: (g, 0, 0)),
            _full_spec((VOCAB, 128)),
            _full_spec((3, 128, 128)), _full_spec((3, 128, 128)),
            _full_spec((3, 128, 128)), _full_spec((3, 128, 128)),
            _full_spec((3, 128)), _full_spec((3, 128)),
            _full_spec((3, 128)), _full_spec((3, 128)),
            _full_spec((3, 128)), _full_spec((3, 128)),
            _full_spec((3, 128)), _full_spec((3, 128)),
            _full_spec((3, 128)), _full_spec((3, 128)),
            _full_spec((3, 128, 128)), _full_spec((3, 128)),
            _full_spec((3, 128, 128)), _full_spec((3, 128)),
            _full_spec((3, 128, 128)), _full_spec((3, 128)),
            _full_spec((3, 128, 128)), _full_spec((3, 128)),
            _full_spec((8, SEQ, 32)), _full_spec((32, 1)),
        ],
        out_specs=pl.BlockSpec((1, 32, 121), lambda g: (g, 0, 0)),
        out_shape=jax.ShapeDtypeStruct((N_GRAPHS, 32, 121), jnp.float32),
    )(tgt3, xg.reshape(N_GRAPHS, 1, 128), emb,
      sd['wq'], sd['wk'], sd['wv'], sd['wo'],
      sd['bq'], sd['bk'], sd['bv'], sd['bo'],
      sd['l1g'], sd['l1b'], sd['l2g'], sd['l2b'], sd['l3g'], sd['l3b'],
      sd['cwv'], sd['cbv'], sd['cwo'], sd['cbo'],
      sd['pw1'], sd['pb1'], sd['pw2'], sd['pb2'],
      cw8, cb)


def _k9_body(cf_ref, xtw_ref, xtb_ref, xg_ref,
             ssf_ref, sasf_ref, edsf_ref,
             w3_ref, b3_ref, w4_ref, b4_ref, w5_ref, b5_ref,
             w1_ref, b1_ref, w2_ref, b2_ref, wo_ref, bo_ref, out_ref):
    xt = jnp.dot(cf_ref[...], xtw_ref[...],
                 preferred_element_type=jnp.float32) + xtb_ref[...]
    rows = lax.broadcasted_iota(jnp.int32, (N_SS, 1), 0)
    sel16 = (rows == lax.broadcasted_iota(jnp.int32, (1, N_GRAPHS), 1)
             ).astype(jnp.float32)
    sel511 = (rows == N_SS - 1).astype(jnp.float32)

    def pool_mat(feat, w, b):
        mx = jnp.max(feat, axis=0, keepdims=True)
        mn = jnp.mean(feat, axis=0, keepdims=True)
        row = jnp.concatenate([mx, mn], axis=1)
        pr = jnp.dot(row, w, preferred_element_type=jnp.float32)
        return jnp.dot(sel511, pr, preferred_element_type=jnp.float32) + b

    ss = pool_mat(ssf_ref[...], w3_ref[...], b3_ref[...])
    sas = pool_mat(sasf_ref[...], w4_ref[...], b4_ref[...])
    eds = pool_mat(edsf_ref[...], w5_ref[...], b5_ref[...])
    xgp = jnp.dot(sel16, xg_ref[...], preferred_element_type=jnp.float32)
    xtp = jnp.dot(sel16, xt, preferred_element_type=jnp.float32)
    w1 = w1_ref[...]
    z = (jnp.dot(xgp, w1[0:128], preferred_element_type=jnp.float32)
         + jnp.dot(xtp, w1[128:256], preferred_element_type=jnp.float32)
         + jnp.dot(ss, w1[256:384], preferred_element_type=jnp.float32)
         + jnp.dot(sas, w1[384:512], preferred_element_type=jnp.float32)
         + jnp.dot(eds, w1[512:640], preferred_element_type=jnp.float32))
    h1 = jnp.maximum(z + b1_ref[...], 0.0)
    h2 = jnp.maximum(jnp.dot(h1, w2_ref[...],
                             preferred_element_type=jnp.float32) + b2_ref[...], 0.0)
    out_ref[...] = jnp.dot(h2, wo_ref[...],
                           preferred_element_type=jnp.float32) + bo_ref[...]


def _tc_final(cflat, xg, ss_feat, sas_feat, eds_contact, p):
    return pl.pallas_call(
        _k9_body,
        out_shape=jax.ShapeDtypeStruct((N_SS, 1), jnp.float32),
    )(cflat, p['fc1_xt_w'], p['fc1_xt_b'][None, :], xg,
      ss_feat, sas_feat, eds_contact,
      p['fc_g3_w'], p['fc_g3_b'][None, :],
      p['fc_g4_w'], p['fc_g4_b'][None, :],
      p['fc_g5_w'], p['fc_g5_b'][None, :],
      p['fc1_w'], p['fc1_b'][None, :], p['fc2_w'], p['fc2_b'][None, :],
      p['out_w'], p['out_b'][None, :])


def kernel(x, edge_index, batch, target, ss_feat, sas_feat, eds_contact, params):
    p = params
    H, C = 10, 78
    f32 = jnp.float32

    src, dst = edge_index[0], edge_index[1]
    srcp = jnp.concatenate([src, jnp.zeros((_EP - _E,), src.dtype)])
    dstp = jnp.concatenate([dst, jnp.full((_EP - _E,), _N, dst.dtype)])
    src2 = srcp.reshape(_EP // 128, 128)
    dst2 = dstp.reshape(_EP // 128, 128)
    src7 = (srcp[None, :] + (_NP * jnp.arange(7, dtype=srcp.dtype))[:, None]
            ).reshape(7, _EP // 128, 128)

    att = p['gat_att']
    blk = jnp.zeros((H, C, H), f32).at[jnp.arange(H), :, jnp.arange(H)].set(att)
    a_m = jnp.pad(blk.reshape(H * C, H), ((0, _P - H * C), (0, 6)))
    rblk = jnp.zeros((H, H, C), f32).at[jnp.arange(H), jnp.arange(H), :].set(1.0)
    r_m = jnp.pad(rblk.reshape(H, H * C), ((0, 6), (0, _P - H * C)))
    pmask = jnp.pad(jnp.zeros((1, H * C), f32), ((0, 0), (0, _P - H * C)),
                    constant_values=1.0)

    x_pad = jnp.pad(x, ((0, _NP - _N), (0, 0)))
    wl_pad = jnp.pad(p['gat_wl'], ((0, 0), (0, _P - H * C)))
    wr_pad = jnp.pad(p['gat_wr'], ((0, 0), (0, _P - H * C)))
    gatb = jnp.pad(p['gat_b'], (0, _P - H * C))[None, :]
    gcnw = jnp.pad(p['gcn_w'], ((0, _P - H * C), (0, _P - H * C)))
    gcnb = jnp.pad(p['gcn_b'], (0, _P - H * C))[None, :]
    fg1 = p['fc_g1_w']
    w1p = jnp.concatenate([
        jnp.pad(fg1[:H * C], ((0, _P - H * C), (0, 0))),
        jnp.pad(fg1[H * C:], ((0, _P - H * C), (0, 0))),
    ], axis=0)
    batchf = jnp.pad(batch.astype(f32), (0, _NP - _N),
                     constant_values=float(N_GRAPHS))[:, None]

    sd = {
        'wq': jnp.stack([L['sa_wq'] for L in p['dec']]),
        'wk': jnp.stack([L['sa_wk'] for L in p['dec']]),
        'wv': jnp.stack([L['sa_wv'] for L in p['dec']]),
        'wo': jnp.stack([L['sa_wo'] for L in p['dec']]),
        'bq': jnp.stack([L['sa_bq'] for L in p['dec']]),
        'bk': jnp.stack([L['sa_bk'] for L in p['dec']]),
        'bv': jnp.stack([L['sa_bv'] for L in p['dec']]),
        'bo': jnp.stack([L['sa_bo'] for L in p['dec']]),
        'l1g': jnp.stack([L['ln1_g'] for L in p['dec']]),
        'l1b': jnp.stack([L['ln1_b'] for L in p['dec']]),
        'l2g': jnp.stack([L['ln2_g'] for L in p['dec']]),
        'l2b': jnp.stack([L['ln2_b'] for L in p['dec']]),
        'l3g': jnp.stack([L['ln3_g'] for L in p['dec']]),
        'l3b': jnp.stack([L['ln3_b'] for L in p['dec']]),
        'cwv': jnp.stack([L['ca_wv'] for L in p['dec']]),
        'cbv': jnp.stack([L['ca_bv'] for L in p['dec']]),
        'cwo': jnp.stack([L['ca_wo'] for L in p['dec']]),
        'cbo': jnp.stack([L['ca_bo'] for L in p['dec']]),
        'pw1': jnp.stack([L['pf_w1'] for L in p['dec']]),
        'pb1': jnp.stack([L['pf_b1'] for L in p['dec']]),
        'pw2': jnp.stack([L['pf_w2'] for L in p['dec']]),
        'pb2': jnp.stack([L['pf_b2'] for L in p['dec']]),
    }
    cw8 = p['conv_w'].transpose(2, 1, 0)
    cb = p['conv_b'][:, None]
    tgt3 = target.astype(jnp.int32)[:, None, :]

    xl_t, xr_t, wself = _tc_project(x_pad, wl_pad, wr_pad, a_m)
    g1, g2 = _sc_gather_pair(xl_t, xr_t, src2, dst2)
    z8 = _tc_edge_math(g1, g2, a_m, r_m)
    part = _sc_scatter_z(z8, dst2)

    q3, st, dv = _tc_gat_assemble(part, xl_t, xr_t, wself, r_m, gatb, pmask,
                                  gcnw, gcnb)
    qflat = q3.reshape(7 * _NP, 128)

    part2 = _sc_gcn_agg(qflat, src7, dst2)
    gmax, gsum, cnt = _tc_pool(part2, st, dv, batchf)
    xg = _tc_graph_head(gmax, gsum, cnt, w1p, p['fc_g1_b'][None, :],
                        p['fc_g2_w'], p['fc_g2_b'][None, :])

    conv16 = _tc_decoder(tgt3, xg, p['emb'], sd, cw8, cb)
    cflat = conv16.reshape(N_GRAPHS, 32 * 121)
    return _tc_final(cflat, xg, ss_feat, sas_feat, eds_contact, p)


# speedup vs baseline: 7.2809x; 1.0399x over previous
import jax
import jax.numpy as jnp
from jax import lax
from jax.experimental import pallas as pl
from jax.experimental.pallas import tpu as pltpu
from jax.experimental.pallas import tpu_sc as plsc

N_GRAPHS = 16
N_SS = 512
SEQ = 1000
VOCAB = 26

_N = 10000
_NP = 10240
_E = 160000
_EP = 163840
_PT = 5120
_P = 896
_CH = 64
_NCH = _PT // _CH
_BN = 640
_BE = 512


def _k2_body(xl_hbm, xr_hbm, src2_hbm, dst2_hbm, g1_hbm, g2_hbm,
             srcv, dstv, buf1, buf2, sem1, sem2):
    c = lax.axis_index("c")
    s = lax.axis_index("s")
    w = s * 2 + c
    nrow = _PT // 128
    pltpu.sync_copy(src2_hbm.at[pl.ds(w * nrow, nrow)], srcv)
    pltpu.sync_copy(dst2_hbm.at[pl.ds(w * nrow, nrow)], dstv)

    def body(r, carry):
        k = 2 * r
        d1 = pltpu.async_copy(xl_hbm.at[srcv.at[r, pl.ds(0, _CH)]], buf1, sem1)
        d2 = pltpu.async_copy(xr_hbm.at[dstv.at[r, pl.ds(0, _CH)]], buf2, sem2)
        d1.wait()
        pltpu.sync_copy(buf1, g1_hbm.at[pl.ds(w * _PT + k * _CH, _CH)])
        d1 = pltpu.async_copy(xl_hbm.at[srcv.at[r, pl.ds(_CH, _CH)]], buf1, sem1)
        d2.wait()
        pltpu.sync_copy(buf2, g2_hbm.at[pl.ds(w * _PT + k * _CH, _CH)])
        d2 = pltpu.async_copy(xr_hbm.at[dstv.at[r, pl.ds(_CH, _CH)]], buf2, sem2)
        d1.wait()
        pltpu.sync_copy(buf1, g1_hbm.at[pl.ds(w * _PT + (k + 1) * _CH, _CH)])
        d2.wait()
        pltpu.sync_copy(buf2, g2_hbm.at[pl.ds(w * _PT + (k + 1) * _CH, _CH)])
        return carry

    lax.fori_loop(0, _NCH // 2, body, 0)


def _sc_gather_pair(xl_t, xr_t, src2, dst2):
    mesh = plsc.VectorSubcoreMesh(core_axis_name="c", subcore_axis_name="s")
    f = pl.kernel(
        _k2_body,
        out_type=(jax.ShapeDtypeStruct((_EP, _P), jnp.float32),
                  jax.ShapeDtypeStruct((_EP, _P), jnp.float32)),
        mesh=mesh,
        scratch_types=[
            pltpu.VMEM((_PT // 128, 128), jnp.int32),
            pltpu.VMEM((_PT // 128, 128), jnp.int32),
            pltpu.VMEM((_CH, _P), jnp.float32),
            pltpu.VMEM((_CH, _P), jnp.float32),
            pltpu.SemaphoreType.DMA,
            pltpu.SemaphoreType.DMA,
        ],
    )
    return f(xl_t, xr_t, src2, dst2)


def _k3_body(g1_ref, g2_ref, a_ref, r_ref, z8_ref):
    sv = g1_ref[...] + g2_ref[...]
    m = jnp.where(sv >= 0, sv, 0.2 * sv)
    e = jnp.dot(m, a_ref[...], preferred_element_type=jnp.float32)
    w = jnp.exp(e)
    wx = jnp.dot(w, r_ref[...], preferred_element_type=jnp.float32)
    z = wx * sv
    for q in range(7):
        z8_ref[q] = z[:, 128 * q:128 * (q + 1)]
    z8_ref[7] = jnp.concatenate(
        [w, jnp.zeros((w.shape[0], 112), jnp.float32)], axis=1)


def _tc_edge_math(g1, g2, a_m, r_m):
    return pl.pallas_call(
        _k3_body,
        grid=(_EP // _BE,),
        in_specs=[
            pl.BlockSpec((_BE, _P), lambda i: (i, 0)),
            pl.BlockSpec((_BE, _P), lambda i: (i, 0)),
            pl.BlockSpec((_P, 16), lambda i: (0, 0)),
            pl.BlockSpec((16, _P), lambda i: (0, 0)),
        ],
        out_specs=pl.BlockSpec((8, _BE, 128), lambda i: (0, i, 0)),
        out_shape=jax.ShapeDtypeStruct((8, _EP, 128), jnp.float32),
    )(g1, g2, a_m, r_m)


def _k4_body(z8_hbm, dst2_hbm, part_hbm, dstv, buf, buf2, acc, sem, sem2):
    c = lax.axis_index("c")
    s = lax.axis_index("s")
    w = s * 2 + c

    pltpu.sync_copy(dst2_hbm.at[pl.ds(w * 40, 40)], dstv)

    def zero_row(i, carry):
        for j in range(8):
            buf[i, pl.ds(j * 16, 16)] = jnp.zeros((16,), jnp.float32)
        return carry

    for qp in range(8):
        lax.fori_loop(0, 128, zero_row, 0)
        for j in range(5):
            pltpu.sync_copy(buf, acc.at[pl.ds(s * 640 + j * 128, 128)])
        plsc.subcore_barrier()

        def chunk(r, carry):
            k = 2 * r
            da = pltpu.async_copy(
                z8_hbm.at[qp, pl.ds(w * _PT + k * 128, 128)], buf, sem)
            db = pltpu.async_copy(
                z8_hbm.at[qp, pl.ds(w * _PT + (k + 1) * 128, 128)], buf2, sem2)
            da.wait()
            pltpu.sync_copy(buf, acc.at[dstv.at[k]], add=True)
            db.wait()
            pltpu.sync_copy(buf2, acc.at[dstv.at[k + 1]], add=True)
            return carry

        lax.fori_loop(0, 20, chunk, 0)
        plsc.subcore_barrier()
        pltpu.sync_copy(acc.at[pl.ds(s * 640, 640)],
                        part_hbm.at[c, qp, pl.ds(s * 640, 640)])
        plsc.subcore_barrier()


def _sc_scatter_z(z8, dst2):
    mesh = plsc.VectorSubcoreMesh(core_axis_name="c", subcore_axis_name="s")
    f = pl.kernel(
        _k4_body,
        out_type=jax.ShapeDtypeStruct((2, 8, _NP, 128), jnp.float32),
        mesh=mesh,
        scratch_types=[
            pltpu.VMEM((40, 128), jnp.int32),
            pltpu.VMEM((128, 128), jnp.float32),
            pltpu.VMEM((128, 128), jnp.float32),
            pltpu.VMEM_SHARED((_NP, 128), jnp.float32),
            pltpu.SemaphoreType.DMA,
            pltpu.SemaphoreType.DMA,
        ],
    )
    return f(z8, dst2)


def _k6_body(qflat_hbm, src7_hbm, dst2_hbm, part_hbm, srcv, dstv, buf, buf2, acc, sem, sem2):
    c = lax.axis_index("c")
    s = lax.axis_index("s")
    w = s * 2 + c

    pltpu.sync_copy(dst2_hbm.at[pl.ds(w * 40, 40)], dstv)

    def zero_row(i, carry):
        for j in range(8):
            buf[i, pl.ds(j * 16, 16)] = jnp.zeros((16,), jnp.float32)
        return carry

    for qp in range(7):
        pltpu.sync_copy(src7_hbm.at[qp, pl.ds(w * 40, 40)], srcv)
        lax.fori_loop(0, 128, zero_row, 0)
        for j in range(5):
            pltpu.sync_copy(buf, acc.at[pl.ds(s * 640 + j * 128, 128)])
        plsc.subcore_barrier()

        def chunk(r, carry):
            k = 2 * r
            da = pltpu.async_copy(qflat_hbm.at[srcv.at[k]], buf, sem)
            db = pltpu.async_copy(qflat_hbm.at[srcv.at[k + 1]], buf2, sem2)
            da.wait()
            pltpu.sync_copy(buf, acc.at[dstv.at[k]], add=True)
            db.wait()
            pltpu.sync_copy(buf2, acc.at[dstv.at[k + 1]], add=True)
            return carry

        lax.fori_loop(0, 20, chunk, 0)
        plsc.subcore_barrier()
        pltpu.sync_copy(acc.at[pl.ds(s * 640, 640)],
                        part_hbm.at[c, qp, pl.ds(s * 640, 640)])
        plsc.subcore_barrier()


def _sc_gcn_agg(qflat, src7, dst2):
    mesh = plsc.VectorSubcoreMesh(core_axis_name="c", subcore_axis_name="s")
    f = pl.kernel(
        _k6_body,
        out_type=jax.ShapeDtypeStruct((2, 7, _NP, 128), jnp.float32),
        mesh=mesh,
        scratch_types=[
            pltpu.VMEM((40, 128), jnp.int32),
            pltpu.VMEM((40, 128), jnp.int32),
            pltpu.VMEM((128, 128), jnp.float32),
            pltpu.VMEM((128, 128), jnp.float32),
            pltpu.VMEM_SHARED((_NP, 128), jnp.float32),
            pltpu.SemaphoreType.DMA,
            pltpu.SemaphoreType.DMA,
        ],
    )
    return f(qflat, src7, dst2)


def _k1_body(x_ref, wl_ref, wr_ref, a_ref, xl_ref, xr_ref, ws_ref):
    xb = x_ref[...]
    xl = jnp.dot(xb, wl_ref[...], preferred_element_type=jnp.float32)
    xr = jnp.dot(xb, wr_ref[...], preferred_element_type=jnp.float32)
    sv = xl + xr
    m = jnp.where(sv >= 0, sv, 0.2 * sv)
    ws_ref[...] = jnp.exp(jnp.dot(m, a_ref[...],
                                  preferred_element_type=jnp.float32))
    xl_ref[...] = xl
    xr_ref[...] = xr


def _tc_project(x_pad, wl_pad, wr_pad, a_m):
    return pl.pallas_call(
        _k1_body,
        grid=(_NP // _BN,),
        in_specs=[
            pl.BlockSpec((_BN, 78), lambda i: (i, 0)),
            pl.BlockSpec((78, _P), lambda i: (0, 0)),
            pl.BlockSpec((78, _P), lambda i: (0, 0)),
            pl.BlockSpec((_P, 16), lambda i: (0, 0)),
        ],
        out_specs=[
            pl.BlockSpec((_BN, _P), lambda i: (i, 0)),
            pl.BlockSpec((_BN, _P), lambda i: (i, 0)),
            pl.BlockSpec((_BN, 16), lambda i: (i, 0)),
        ],
        out_shape=[
            jax.ShapeDtypeStruct((_NP, _P), jnp.float32),
            jax.ShapeDtypeStruct((_NP, _P), jnp.float32),
            jax.ShapeDtypeStruct((_NP, 16), jnp.float32),
        ],
    )(x_pad, wl_pad, wr_pad, a_m)


def _k5_body(part_ref, xl_ref, xr_ref, ws_ref, r_ref, gatb_ref, pmask_ref,
             gcnw_ref, gcnb_ref, q3_ref, st_ref, dv_ref):
    pb = part_ref[...]
    acc = jnp.concatenate([pb[0, q] + pb[1, q] for q in range(7)], axis=1)
    den_e = (pb[0, 7] + pb[1, 7])[:, :16]
    ws = ws_ref[...]
    den_tot = den_e + ws
    rm = r_ref[...]
    corr = (jnp.dot(ws, rm, preferred_element_type=jnp.float32) * xl_ref[...]
            - jnp.dot(den_e, rm, preferred_element_type=jnp.float32) * xr_ref[...])
    denx = (jnp.dot(den_tot, rm, preferred_element_type=jnp.float32)
            + pmask_ref[...] + 1e-16)
    h = jnp.maximum((acc + corr) / denx + gatb_ref[...], 0.0)
    h2 = jnp.dot(h, gcnw_ref[...], preferred_element_type=jnp.float32)
    deg = den_tot[:, 15:16]
    dinv = lax.rsqrt(jnp.maximum(deg, 1.0))
    q = dinv * h2
    for qp in range(7):
        q3_ref[qp] = q[:, 128 * qp:128 * (qp + 1)]
    st_ref[...] = dinv * dinv * h2 + gcnb_ref[...]
    dv_ref[...] = dinv * jnp.ones((1, 8), jnp.float32)


def _tc_gat_assemble(part, xl_t, xr_t, wself, r_m, gatb, pmask, gcnw, gcnb):
    return pl.pallas_call(
        _k5_body,
        grid=(_NP // _BN,),
        in_specs=[
            pl.BlockSpec((2, 8, _BN, 128), lambda i: (0, 0, i, 0)),
            pl.BlockSpec((_BN, _P), lambda i: (i, 0)),
            pl.BlockSpec((_BN, _P), lambda i: (i, 0)),
            pl.BlockSpec((_BN, 16), lambda i: (i, 0)),
            pl.BlockSpec((16, _P), lambda i: (0, 0)),
            pl.BlockSpec((1, _P), lambda i: (0, 0)),
            pl.BlockSpec((1, _P), lambda i: (0, 0)),
            pl.BlockSpec((_P, _P), lambda i: (0, 0)),
            pl.BlockSpec((1, _P), lambda i: (0, 0)),
        ],
        out_specs=[
            pl.BlockSpec((7, _BN, 128), lambda i: (0, i, 0)),
            pl.BlockSpec((_BN, _P), lambda i: (i, 0)),
            pl.BlockSpec((_BN, 8), lambda i: (i, 0)),
        ],
        out_shape=[
            jax.ShapeDtypeStruct((7, _NP, 128), jnp.float32),
            jax.ShapeDtypeStruct((_NP, _P), jnp.float32),
            jax.ShapeDtypeStruct((_NP, 8), jnp.float32),
        ],
    )(part, xl_t, xr_t, wself, r_m, gatb, pmask, gcnw, gcnb)


def _k7_body(part2_ref, st_ref, dv_ref, bf_ref, gmax_ref, gsum_ref, cnt_ref):
    pb = part2_ref[...]
    agg = jnp.concatenate([pb[0, q] + pb[1, q] for q in range(7)], axis=1)
    hg = jnp.maximum(dv_ref[...][:, :1] * agg + st_ref[...], 0.0)
    bcol = bf_ref[...]
    gm, gs, ct = [], [], []
    for g in range(N_GRAPHS):
        mk = (bcol == float(g)).astype(jnp.float32)
        sel = hg * mk
        gm.append(jnp.max(sel, axis=0, keepdims=True))
        gs.append(jnp.sum(sel, axis=0, keepdims=True))
        ct.append(jnp.sum(mk, axis=0, keepdims=True))
    gm = jnp.concatenate(gm, axis=0)
    gs = jnp.concatenate(gs, axis=0)
    ct = jnp.concatenate(ct, axis=0) * jnp.ones((1, 128), jnp.float32)
    first = pl.program_id(0) == 0
    gmax_ref[...] = jnp.where(first, gm, jnp.maximum(gmax_ref[...], gm))
    gsum_ref[...] = jnp.where(first, gs, gsum_ref[...] + gs)
    cnt_ref[...] = jnp.where(first, ct, cnt_ref[...] + ct)


def _tc_pool(part2, st, dv, batchf):
    return pl.pallas_call(
        _k7_body,
        grid=(_NP // _BN,),
        in_specs=[
            pl.BlockSpec((2, 7, _BN, 128), lambda i: (0, 0, i, 0)),
            pl.BlockSpec((_BN, _P), lambda i: (i, 0)),
            pl.BlockSpec((_BN, 8), lambda i: (i, 0)),
            pl.BlockSpec((_BN, 1), lambda i: (i, 0)),
        ],
        out_specs=[
            pl.BlockSpec((N_GRAPHS, _P), lambda i: (0, 0)),
            pl.BlockSpec((N_GRAPHS, _P), lambda i: (0, 0)),
            pl.BlockSpec((N_GRAPHS, 128), lambda i: (0, 0)),
        ],
        out_shape=[
            jax.ShapeDtypeStruct((N_GRAPHS, _P), jnp.float32),
            jax.ShapeDtypeStruct((N_GRAPHS, _P), jnp.float32),
            jax.ShapeDtypeStruct((N_GRAPHS, 128), jnp.float32),
        ],
    )(part2, st, dv, batchf)


def _k7b_body(gmax_ref, gsum_ref, cnt_ref, w1_ref, b1_ref, w2_ref, b2_ref,
              xg_ref):
    gap = gsum_ref[...] / jnp.maximum(cnt_ref[...][:, :1], 1.0)
    xcat = jnp.concatenate([gmax_ref[...], gap], axis=1)
    h1 = jnp.maximum(jnp.dot(xcat, w1_ref[...],
                             preferred_element_type=jnp.float32) + b1_ref[...], 0.0)
    xg_ref[...] = jnp.dot(h1, w2_ref[...],
                          preferred_element_type=jnp.float32) + b2_ref[...]


def _tc_graph_head(gmax, gsum, cnt, w1p, b1, w2, b2):
    return pl.pallas_call(
        _k7b_body,
        out_shape=jax.ShapeDtypeStruct((N_GRAPHS, 128), jnp.float32),
    )(gmax, gsum, cnt, w1p, b1, w2, b2)


def _ln_p(v, g, b):
    mu = jnp.mean(v, axis=-1, keepdims=True)
    d = v - mu
    var = jnp.mean(d * d, axis=-1, keepdims=True)
    return d * lax.rsqrt(var + 1e-5) * g + b


def _k8_body(tgt_ref, xg_ref, emb_ref,
             saq_ref, sak_ref, sav_ref, sao_ref,
             sabq_ref, sabk_ref, sabv_ref, sabo_ref,
             ln1g_ref, ln1b_ref, ln2g_ref, ln2b_ref, ln3g_ref, ln3b_ref,
             cawv_ref, cabv_ref, cawo_ref, cabo_ref,
             pfw1_ref, pfb1_ref, pfw2_ref, pfb2_ref,
             cw_ref, cb_ref, conv_ref):
    tgt = tgt_ref[...].reshape(1, SEQ)
    ohT = (tgt == lax.broadcasted_iota(jnp.int32, (VOCAB, 1), 0)
           ).astype(jnp.float32)
    t = lax.dot_general(ohT, emb_ref[...], (((0,), (0,)), ((), ())),
                        preferred_element_type=jnp.float32)
    xgr = xg_ref[...].reshape(1, 128)
    for i in range(3):
        q = jnp.dot(t, saq_ref[i], preferred_element_type=jnp.float32) + sabq_ref[i]
        k = jnp.dot(t, sak_ref[i], preferred_element_type=jnp.float32) + sabk_ref[i]
        v = jnp.dot(t, sav_ref[i], preferred_element_type=jnp.float32) + sabv_ref[i]
        outs = []
        for hh in range(2):
            qh = q[:, 64 * hh:64 * hh + 64]
            kh = k[:, 64 * hh:64 * hh + 64]
            vh = v[:, 64 * hh:64 * hh + 64]
            sc = lax.dot_general(qh, kh, (((1,), (1,)), ((), ())),
                                 preferred_element_type=jnp.float32) * 0.125
            mx = jnp.max(sc, axis=-1, keepdims=True)
            pe = jnp.exp(sc - mx)
            aw = pe / jnp.sum(pe, axis=-1, keepdims=True)
            outs.append(jnp.dot(aw, vh, preferred_element_type=jnp.float32))
        o = jnp.dot(jnp.concatenate(outs, axis=1), sao_ref[i],
                    preferred_element_type=jnp.float32) + sabo_ref[i]
        t = _ln_p(t + o, ln1g_ref[i], ln1b_ref[i])
        ca = jnp.dot(jnp.dot(xgr, cawv_ref[i],
                             preferred_element_type=jnp.float32) + cabv_ref[i],
                     cawo_ref[i], preferred_element_type=jnp.float32) + cabo_ref[i]
        t = _ln_p(t + ca, ln2g_ref[i], ln2b_ref[i])
        f = jnp.dot(jnp.maximum(jnp.dot(t, pfw1_ref[i],
                                        preferred_element_type=jnp.float32)
                                + pfb1_ref[i], 0.0),
                    pfw2_ref[i], preferred_element_type=jnp.float32) + pfb2_ref[i]
        t = _ln_p(t + f, ln3g_ref[i], ln3b_ref[i])
    acc = cb_ref[...] * jnp.ones((1, 121), jnp.float32)
    for k8 in range(8):
        acc = acc + lax.dot_general(
            cw_ref[k8], t[:, k8:k8 + 121], (((0,), (0,)), ((), ())),
            preferred_element_type=jnp.float32)
    conv_ref[0] = acc


def _full_spec(shape):
    nd = len(shape)
    return pl.BlockSpec(shape, lambda g, _nd=nd: (0,) * _nd)


def _tc_decoder(tgt3, xg, emb, sd, cw8, cb):
    return pl.pallas_call(
        _k8_body,
        grid=(N_GRAPHS,),
        in_specs=[
            pl.BlockSpec((1, 1, SEQ), lambda g: (g, 0, 0)),
            pl.BlockSpec((1, 1, 128), lambda g: (g, 0, 0)),
            _full_spec((VOCAB, 128)),
            _full_spec((3, 128, 128)), _full_spec((3, 128, 128)),
            _full_spec((3, 128, 128)), _full_spec((3, 128, 128)),
            _full_spec((3, 128)), _full_spec((3, 128)),
            _full_spec((3, 128)), _full_spec((3, 128)),
            _full_spec((3, 128)), _full_spec((3, 128)),
            _full_spec((3, 128)), _full_spec((3, 128)),
            _full_spec((3, 128)), _full_spec((3, 128)),
            _full_spec((3, 128, 128)), _full_spec((3, 128)),
            _full_spec((3, 128, 128)), _full_spec((3, 128)),
            _full_spec((3, 128, 128)), _full_spec((3, 128)),
            _full_spec((3, 128, 128)), _full_spec((3, 128)),
            _full_spec((8, SEQ, 32)), _full_spec((32, 1)),
        ],
        out_specs=pl.BlockSpec((1, 32, 121), lambda g: (g, 0, 0)),
        out_shape=jax.ShapeDtypeStruct((N_GRAPHS, 32, 121), jnp.float32),
    )(tgt3, xg.reshape(N_GRAPHS, 1, 128), emb,
      sd['wq'], sd['wk'], sd['wv'], sd['wo'],
      sd['bq'], sd['bk'], sd['bv'], sd['bo'],
      sd['l1g'], sd['l1b'], sd['l2g'], sd['l2b'], sd['l3g'], sd['l3b'],
      sd['cwv'], sd['cbv'], sd['cwo'], sd['cbo'],
      sd['pw1'], sd['pb1'], sd['pw2'], sd['pb2'],
      cw8, cb)


def _k9_body(cf_ref, xtw_ref, xtb_ref, xg_ref,
             ssf_ref, sasf_ref, edsf_ref,
             w3_ref, b3_ref, w4_ref, b4_ref, w5_ref, b5_ref,
             w1_ref, b1_ref, w2_ref, b2_ref, wo_ref, bo_ref, out_ref):
    xt = jnp.dot(cf_ref[...], xtw_ref[...],
                 preferred_element_type=jnp.float32) + xtb_ref[...]
    rows = lax.broadcasted_iota(jnp.int32, (N_SS, 1), 0)
    sel16 = (rows == lax.broadcasted_iota(jnp.int32, (1, N_GRAPHS), 1)
             ).astype(jnp.float32)
    sel511 = (rows == N_SS - 1).astype(jnp.float32)

    def pool_mat(feat, w, b):
        mx = jnp.max(feat, axis=0, keepdims=True)
        mn = jnp.mean(feat, axis=0, keepdims=True)
        row = jnp.concatenate([mx, mn], axis=1)
        pr = jnp.dot(row, w, preferred_element_type=jnp.float32)
        return jnp.dot(sel511, pr, preferred_element_type=jnp.float32) + b

    ss = pool_mat(ssf_ref[...], w3_ref[...], b3_ref[...])
    sas = pool_mat(sasf_ref[...], w4_ref[...], b4_ref[...])
    eds = pool_mat(edsf_ref[...], w5_ref[...], b5_ref[...])
    xgp = jnp.dot(sel16, xg_ref[...], preferred_element_type=jnp.float32)
    xtp = jnp.dot(sel16, xt, preferred_element_type=jnp.float32)
    w1 = w1_ref[...]
    z = (jnp.dot(xgp, w1[0:128], preferred_element_type=jnp.float32)
         + jnp.dot(xtp, w1[128:256], preferred_element_type=jnp.float32)
         + jnp.dot(ss, w1[256:384], preferred_element_type=jnp.float32)
         + jnp.dot(sas, w1[384:512], preferred_element_type=jnp.float32)
         + jnp.dot(eds, w1[512:640], preferred_element_type=jnp.float32))
    h1 = jnp.maximum(z + b1_ref[...], 0.0)
    h2 = jnp.maximum(jnp.dot(h1, w2_ref[...],
                             preferred_element_type=jnp.float32) + b2_ref[...], 0.0)
    out_ref[...] = jnp.dot(h2, wo_ref[...],
                           preferred_element_type=jnp.float32) + bo_ref[...]


def _tc_final(cflat, xg, ss_feat, sas_feat, eds_contact, p):
    return pl.pallas_call(
        _k9_body,
        out_shape=jax.ShapeDtypeStruct((N_SS, 1), jnp.float32),
    )(cflat, p['fc1_xt_w'], p['fc1_xt_b'][None, :], xg,
      ss_feat, sas_feat, eds_contact,
      p['fc_g3_w'], p['fc_g3_b'][None, :],
      p['fc_g4_w'], p['fc_g4_b'][None, :],
      p['fc_g5_w'], p['fc_g5_b'][None, :],
      p['fc1_w'], p['fc1_b'][None, :], p['fc2_w'], p['fc2_b'][None, :],
      p['out_w'], p['out_b'][None, :])


def kernel(x, edge_index, batch, target, ss_feat, sas_feat, eds_contact, params):
    p = params
    H, C = 10, 78
    f32 = jnp.float32

    src, dst = edge_index[0], edge_index[1]
    srcp = jnp.concatenate([src, jnp.zeros((_EP - _E,), src.dtype)])
    dstp = jnp.concatenate([dst, jnp.full((_EP - _E,), _N, dst.dtype)])
    src2 = srcp.reshape(_EP // 128, 128)
    dst2 = dstp.reshape(_EP // 128, 128)
    src7 = (srcp[None, :] + (_NP * jnp.arange(7, dtype=srcp.dtype))[:, None]
            ).reshape(7, _EP // 128, 128)

    att = p['gat_att']
    blk = jnp.zeros((H, C, H), f32).at[jnp.arange(H), :, jnp.arange(H)].set(att)
    a_m = jnp.pad(blk.reshape(H * C, H), ((0, _P - H * C), (0, 6)))
    rblk = jnp.zeros((H, H, C), f32).at[jnp.arange(H), jnp.arange(H), :].set(1.0)
    r_m = jnp.pad(rblk.reshape(H, H * C), ((0, 6), (0, _P - H * C)))
    pmask = jnp.pad(jnp.zeros((1, H * C), f32), ((0, 0), (0, _P - H * C)),
                    constant_values=1.0)

    x_pad = jnp.pad(x, ((0, _NP - _N), (0, 0)))
    wl_pad = jnp.pad(p['gat_wl'], ((0, 0), (0, _P - H * C)))
    wr_pad = jnp.pad(p['gat_wr'], ((0, 0), (0, _P - H * C)))
    gatb = jnp.pad(p['gat_b'], (0, _P - H * C))[None, :]
    gcnw = jnp.pad(p['gcn_w'], ((0, _P - H * C), (0, _P - H * C)))
    gcnb = jnp.pad(p['gcn_b'], (0, _P - H * C))[None, :]
    fg1 = p['fc_g1_w']
    w1p = jnp.concatenate([
        jnp.pad(fg1[:H * C], ((0, _P - H * C), (0, 0))),
        jnp.pad(fg1[H * C:], ((0, _P - H * C), (0, 0))),
    ], axis=0)
    batchf = jnp.pad(batch.astype(f32), (0, _NP - _N),
                     constant_values=float(N_GRAPHS))[:, None]

    sd = {
        'wq': jnp.stack([L['sa_wq'] for L in p['dec']]),
        'wk': jnp.stack([L['sa_wk'] for L in p['dec']]),
        'wv': jnp.stack([L['sa_wv'] for L in p['dec']]),
        'wo': jnp.stack([L['sa_wo'] for L in p['dec']]),
        'bq': jnp.stack([L['sa_bq'] for L in p['dec']]),
        'bk': jnp.stack([L['sa_bk'] for L in p['dec']]),
        'bv': jnp.stack([L['sa_bv'] for L in p['dec']]),
        'bo': jnp.stack([L['sa_bo'] for L in p['dec']]),
        'l1g': jnp.stack([L['ln1_g'] for L in p['dec']]),
        'l1b': jnp.stack([L['ln1_b'] for L in p['dec']]),
        'l2g': jnp.stack([L['ln2_g'] for L in p['dec']]),
        'l2b': jnp.stack([L['ln2_b'] for L in p['dec']]),
        'l3g': jnp.stack([L['ln3_g'] for L in p['dec']]),
        'l3b': jnp.stack([L['ln3_b'] for L in p['dec']]),
        'cwv': jnp.stack([L['ca_wv'] for L in p['dec']]),
        'cbv': jnp.stack([L['ca_bv'] for L in p['dec']]),
        'cwo': jnp.stack([L['ca_wo'] for L in p['dec']]),
        'cbo': jnp.stack([L['ca_bo'] for L in p['dec']]),
        'pw1': jnp.stack([L['pf_w1'] for L in p['dec']]),
        'pb1': jnp.stack([L['pf_b1'] for L in p['dec']]),
        'pw2': jnp.stack([L['pf_w2'] for L in p['dec']]),
        'pb2': jnp.stack([L['pf_b2'] for L in p['dec']]),
    }
    cw8 = p['conv_w'].transpose(2, 1, 0)
    cb = p['conv_b'][:, None]
    tgt3 = target.astype(jnp.int32)[:, None, :]

    xl_t, xr_t, wself = _tc_project(x_pad, wl_pad, wr_pad, a_m)
    g1, g2 = _sc_gather_pair(xl_t, xr_t, src2, dst2)
    z8 = _tc_edge_math(g1, g2, a_m, r_m)
    part = _sc_scatter_z(z8, dst2)

    q3, st, dv = _tc_gat_assemble(part, xl_t, xr_t, wself, r_m, gatb, pmask,
                                  gcnw, gcnb)
    qflat = q3.reshape(7 * _NP, 128)

    part2 = _sc_gcn_agg(qflat, src7, dst2)
    gmax, gsum, cnt = _tc_pool(part2, st, dv, batchf)
    xg = _tc_graph_head(gmax, gsum, cnt, w1p, p['fc_g1_b'][None, :],
                        p['fc_g2_w'], p['fc_g2_b'][None, :])

    conv16 = _tc_decoder(tgt3, xg, p['emb'], sd, cw8, cb)
    cflat = conv16.reshape(N_GRAPHS, 32 * 121)
    return _tc_final(cflat, xg, ss_feat, sas_feat, eds_contact, p)
